# Initial kernel scaffold; baseline (speedup 1.0000x reference)
import jax
import jax.numpy as jnp
from jax.experimental import pallas as pl


def kernel(neighbor_index, vertices, feature_map, weights, bias, directions):
    raise NotImplementedError("write your pallas kernel here")


# speedup vs baseline: 3.4442x; 3.4442x over previous
import functools

import jax
import jax.numpy as jnp
from jax import lax
from jax.experimental import pallas as pl
from jax.experimental.pallas import tpu as pltpu
from jax.experimental.pallas import tpu_sc as plsc

IN_C = 512
OUT_C = 512
NBR = 16
V = 10000

NC, NS, L = 2, 16, 16
NW = NC * NS
CHUNK = 320
VP = NW * CHUNK
CB = 4
NBLK = CHUNK // CB
NCH = OUT_C // L


def _mm_body(fm_ref, w_ref, b_ref, dirs_ref, cen_ref, sup_ref, dn_ref):
    fo = jnp.dot(fm_ref[...], w_ref[...], preferred_element_type=jnp.float32)
    fo = fo + b_ref[...]
    cen_ref[...] = fo[:, :OUT_C]
    sup_ref[...] = fo[:, OUT_C:]
    d = dirs_ref[...]
    s = jnp.sum(d * d, axis=0, keepdims=True)
    inv = 1.0 / jnp.maximum(jnp.sqrt(s), 1e-12)
    dn_ref[...] = d * inv


def _tc_stage(fm_p, weights, bias, dirs_p):
    BM = 512
    grid = (VP // BM,)
    return pl.pallas_call(
        _mm_body,
        grid=grid,
        in_specs=[
            pl.BlockSpec((BM, IN_C), lambda i: (i, 0)),
            pl.BlockSpec((IN_C, 2 * OUT_C), lambda i: (0, 0)),
            pl.BlockSpec((1, 2 * OUT_C), lambda i: (0, 0)),
            pl.BlockSpec((8, OUT_C), lambda i: (0, 0)),
        ],
        out_specs=[
            pl.BlockSpec((BM, OUT_C), lambda i: (i, 0)),
            pl.BlockSpec((BM, OUT_C), lambda i: (i, 0)),
            pl.BlockSpec((8, OUT_C), lambda i: (0, 0)),
        ],
        out_shape=[
            jax.ShapeDtypeStruct((VP, OUT_C), jnp.float32),
            jax.ShapeDtypeStruct((VP, OUT_C), jnp.float32),
            jax.ShapeDtypeStruct((8, OUT_C), jnp.float32),
        ],
    )(fm_p, weights, bias, dirs_p)


def _rsqrt_nr(s):
    bits = plsc.bitcast(s, jnp.int32)
    y = plsc.bitcast(jnp.int32(0x5F3759DF) - (bits >> 1), jnp.float32)
    for _ in range(3):
        y = y * (1.5 - 0.5 * s * y * y)
    return y


def _bcast_lane(vec, n):
    return vec.at[jnp.full((L,), n, jnp.int32)].get(mode="promise_in_bounds")


def _sc_body(nbr_hbm, verts_hbm, sup_hbm, cen_hbm, dn_hbm, out_hbm,
             verts_v, dn_v, idx_v, rows_v, cen_v, out_v, sem):
    wid = lax.axis_index("s") * NC + lax.axis_index("c")
    base = wid * CHUNK
    pltpu.sync_copy(verts_hbm, verts_v)
    pltpu.sync_copy(dn_hbm, dn_v)

    def blk_body(b, carry):
        vbase = base + b * CB
        pltpu.sync_copy(nbr_hbm.at[pl.ds(vbase * NBR, CB * NBR)], idx_v)
        pltpu.async_copy(sup_hbm.at[idx_v], rows_v, sem).wait()
        pltpu.sync_copy(cen_hbm.at[pl.ds(vbase, CB)], cen_v)

        for i in range(CB):
            nidx = idx_v[pl.ds(i * NBR, NBR)]
            n3 = nidx * 3
            nx = plsc.load_gather(verts_v, [n3])
            ny = plsc.load_gather(verts_v, [n3 + 1])
            nz = plsc.load_gather(verts_v, [n3 + 2])
            vg = jnp.full((L,), (vbase + i) * 3, jnp.int32)
            cx = plsc.load_gather(verts_v, [vg])
            cy = plsc.load_gather(verts_v, [vg + 1])
            cz = plsc.load_gather(verts_v, [vg + 2])
            dx = nx - cx
            dy = ny - cy
            dz = nz - cz
            s = dx * dx + dy * dy + dz * dz
            y = _rsqrt_nr(s)
            inv = 1.0 / jnp.maximum(s * y, 1e-12)
            dnx = dx * inv
            dny = dy * inv
            dnz = dz * inv

            def ch_body(j, c2):
                sl = pl.ds(j * L, L)
                d0 = dn_v[0, sl]
                d1 = dn_v[1, sl]
                d2 = dn_v[2, sl]
                acc = jnp.full((L,), -jnp.inf, jnp.float32)
                for n in range(NBR):
                    t = (_bcast_lane(dnx, n) * d0
                         + _bcast_lane(dny, n) * d1
                         + _bcast_lane(dnz, n) * d2)
                    t = t / (1.0 + jnp.exp(-t))
                    fs = rows_v[i * NBR + n, sl]
                    acc = jnp.maximum(acc, t * fs)
                out_v[i, sl] = acc + cen_v[i, sl]
                return c2

            lax.fori_loop(0, NCH, ch_body, 0)

        pltpu.sync_copy(out_v, out_hbm.at[pl.ds(vbase, CB)])
        return carry

    lax.fori_loop(0, NBLK, blk_body, 0)


def _sc_stage(nbr_flat, verts_p, support, center, dnorm):
    mesh = plsc.VectorSubcoreMesh(core_axis_name="c", subcore_axis_name="s")
    f = functools.partial(
        pl.kernel,
        out_type=jax.ShapeDtypeStruct((VP, OUT_C), jnp.float32),
        mesh=mesh,
        compiler_params=pltpu.CompilerParams(needs_layout_passes=False),
        scratch_types=[
            pltpu.VMEM((VP * 3,), jnp.float32),
            pltpu.VMEM((8, OUT_C), jnp.float32),
            pltpu.VMEM((CB * NBR,), jnp.int32),
            pltpu.VMEM((CB * NBR, OUT_C), jnp.float32),
            pltpu.VMEM((CB, OUT_C), jnp.float32),
            pltpu.VMEM((CB, OUT_C), jnp.float32),
            pltpu.SemaphoreType.DMA,
        ],
    )(_sc_body)
    return f(nbr_flat, verts_p, support, center, dnorm)


def kernel(neighbor_index, vertices, feature_map, weights, bias, directions):
    bs, v, n = neighbor_index.shape
    nbr = neighbor_index.reshape(v, n).astype(jnp.int32)
    nbr_p = jnp.pad(nbr, ((0, VP - v), (0, 0))).reshape(VP * NBR)
    verts_p = jnp.pad(vertices.reshape(v, 3), ((0, VP - v), (0, 0)),
                      constant_values=1.0).reshape(VP * 3)
    fm_p = jnp.pad(feature_map.reshape(v, IN_C), ((0, VP - v), (0, 0)))
    dirs_p = jnp.pad(directions, ((0, 5), (0, 0)))
    bias2 = bias.reshape(1, 2 * OUT_C)

    center, support, dnorm = _tc_stage(fm_p, weights, bias2, dirs_p)
    out = _sc_stage(nbr_p, verts_p, support, center, dnorm)
    return out[:v].reshape(bs, v, OUT_C)
